# baseline (speedup 1.0000x reference)
import jax
import jax.numpy as jnp
from jax import lax
from jax.experimental import pallas as pl
from jax.experimental.pallas import tpu as pltpu
from jax.experimental.pallas import tpu_sc as plsc

_N = 10000
_E = 320000
_D = 128
_B = 64
_OUT = 24

_NC = 2
_NS = 16
_NW = _NC * _NS
_EW = _E // _NW
_K = 50
_NCH = _EW // _K
_SG = 25
_NB = 5
_NG = _NCH // _SG
_RPT = _N // _NS

_f32 = jnp.float32

_MESH = plsc.VectorSubcoreMesh(core_axis_name="c", subcore_axis_name="s",
                               num_cores=_NC, num_subcores=_NS)


def _zero_fill(buf, nrows):
    def zrow(i, _):
        def zcol(j, _):
            buf[i, pl.ds(j * 16, 16)] = jnp.zeros((16,), _f32)
            return 0
        return lax.fori_loop(0, _D // 16, zcol, 0)
    lax.fori_loop(0, nrows, zrow, 0)


def _zero_acc_slice(rows, acc, r0):
    nfull, tail = _RPT // _K, _RPT % _K
    for t in range(nfull):
        pltpu.sync_copy(rows, acc.at[pl.ds(r0 + t * _K, _K)])
    if tail:
        pltpu.sync_copy(rows.at[pl.ds(0, tail)],
                        acc.at[pl.ds(r0 + nfull * _K, tail)])


def _make_sc_agg():
    def body(*refs):
        (h_hbm, src_hbm, dst_hbm, out_acc) = refs[:4]
        rows = refs[4:4 + _NB]
        sidx, didx, acc = refs[4 + _NB:7 + _NB]
        gsem = refs[7 + _NB:7 + 2 * _NB]
        ssem = refs[7 + 2 * _NB:7 + 3 * _NB]
        rows0 = rows[0]
        cid = lax.axis_index("c")
        sid = lax.axis_index("s")
        wid = cid * _NS + sid
        r0 = sid * _RPT

        _zero_fill(rows0, _K)
        _zero_acc_slice(rows0, acc, r0)

        plsc.subcore_barrier()

        def group(g, _):
            pltpu.sync_copy(src_hbm.at[wid, g], sidx)
            pltpu.sync_copy(dst_hbm.at[wid, g], didx)
            for pb in range(_NB - 1):
                pltpu.async_copy(h_hbm.at[sidx.at[pb]], rows[pb], gsem[pb])

            def step(c, _):
                def consume(b):
                    p = (b + _NB - 1) % _NB
                    pltpu.make_async_copy(
                        h_hbm.at[sidx.at[c]], rows[b], gsem[b]).wait()

                    @pl.when(c >= 1)
                    def _drain():
                        pltpu.make_async_copy(
                            rows[p], acc.at[didx.at[c]], ssem[p]).wait()

                    @pl.when(c + _NB - 1 < _SG)
                    def _prefetch():
                        pltpu.async_copy(
                            h_hbm.at[sidx.at[c + _NB - 1]], rows[p], gsem[p])
                    pltpu.async_copy(rows[b], acc.at[didx.at[c]], ssem[b],
                                     add=True)

                for b in range(_NB):
                    @pl.when(c % _NB == b)
                    def _go(b=b):
                        consume(b)
                return 0
            lax.fori_loop(0, _SG, step, 0)
            last = (_SG - 1) % _NB
            pltpu.make_async_copy(
                rows[last], acc.at[didx.at[0]], ssem[last]).wait()
            return 0
        lax.fori_loop(0, _NG, group, 0)

        plsc.subcore_barrier()
        pltpu.sync_copy(acc.at[pl.ds(r0, _RPT)], out_acc.at[cid, sid])

    return pl.kernel(
        body,
        out_type=[jax.ShapeDtypeStruct((_NC, _NS, _RPT, _D), _f32)],
        mesh=_MESH,
        scratch_types=(
            [pltpu.VMEM((_K, _D), _f32)] * _NB
            + [pltpu.VMEM((_SG, _K), jnp.int32),
               pltpu.VMEM((_SG, _K), jnp.int32),
               pltpu.VMEM_SHARED((_N, _D), _f32)]
            + [pltpu.SemaphoreType.DMA] * (2 * _NB)))


def _make_sc_cnt():
    def body(dst_hbm, out_cnt, ones_b, didx, acc, sem):
        cid = lax.axis_index("c")
        sid = lax.axis_index("s")
        wid = cid * _NS + sid
        r0 = sid * _RPT

        _zero_fill(ones_b, _K)
        _zero_acc_slice(ones_b, acc, r0)

        def orow(i, _):
            def ocol(j, _):
                ones_b[i, pl.ds(j * 16, 16)] = jnp.ones((16,), _f32)
                return 0
            return lax.fori_loop(0, _D // 16, ocol, 0)
        lax.fori_loop(0, _K, orow, 0)
        plsc.subcore_barrier()

        def group(g, _):
            pltpu.sync_copy(dst_hbm.at[wid, g], didx)

            def step(c, _):
                pltpu.async_copy(ones_b, acc.at[didx.at[c]], sem, add=True)

                @pl.when(c >= 3)
                def _drain():
                    pltpu.make_async_copy(
                        ones_b, acc.at[didx.at[c]], sem).wait()
                return 0
            lax.fori_loop(0, _SG, step, 0)
            for _t in range(3):
                pltpu.make_async_copy(
                    ones_b, acc.at[didx.at[0]], sem).wait()
            return 0
        lax.fori_loop(0, _NG, group, 0)

        plsc.subcore_barrier()
        pltpu.sync_copy(acc.at[pl.ds(r0, _RPT)], out_cnt.at[cid, sid])

    return pl.kernel(
        body,
        out_type=jax.ShapeDtypeStruct((_NC, _NS, _RPT, _D), _f32),
        mesh=_MESH,
        scratch_types=[
            pltpu.VMEM((_K, _D), _f32),
            pltpu.VMEM((_SG, _K), jnp.int32),
            pltpu.VMEM_SHARED((_N, _D), _f32),
            pltpu.SemaphoreType.DMA,
        ])


_BN = 1000


def _inv_count(c0, c1):
    return 1.0 / jnp.maximum(c0[:, 0:1] + c1[:, 0:1], 1.0)


def _tc_layer1_body(x, p0, p1, c0, c1, w1, w2, b, o):
    mean = (p0[...] + p1[...]) * _inv_count(c0[...], c1[...])
    h = (jnp.dot(x[...], w1[...], preferred_element_type=_f32)
         + jnp.dot(mean, w2[...], preferred_element_type=_f32) + b[...])
    o[...] = jnp.maximum(h, 0.0)


def _tc_layer2_body(h1, p0, p1, c0, c1, bat, w1, w2, b, wl1, bl1, wl2, bl2,
                    o, psum, pcnt):
    i = pl.program_id(0)

    @pl.when(i == 0)
    def _init():
        psum[...] = jnp.zeros((_B, _D), _f32)
        pcnt[...] = jnp.zeros((_B, 1), _f32)

    mean = (p0[...] + p1[...]) * _inv_count(c0[...], c1[...])
    h2 = (jnp.dot(h1[...], w1[...], preferred_element_type=_f32)
          + jnp.dot(mean, w2[...], preferred_element_type=_f32) + b[...])
    h2 = jnp.maximum(h2, 0.0)

    onehot = (bat[...] == lax.broadcasted_iota(jnp.int32, (_BN, _B), 1)
              ).astype(_f32)
    psum[...] += lax.dot_general(
        onehot, h2, (((0,), (0,)), ((), ())), preferred_element_type=_f32)
    pcnt[...] += jnp.sum(onehot, axis=0).reshape(_B, 1)

    @pl.when(i == pl.num_programs(0) - 1)
    def _head():
        pooled = psum[...] / jnp.maximum(pcnt[...], 1.0)
        z = jnp.maximum(
            jnp.dot(pooled, wl1[...], preferred_element_type=_f32) + bl1[...],
            0.0)
        o[...] = jnp.dot(z, wl2[...], preferred_element_type=_f32) + bl2[...]


def _row_spec(w):
    return pl.BlockSpec((_BN, w), lambda i: (i, 0))


def _full_spec(shape):
    return pl.BlockSpec(shape, lambda i: (0,) * len(shape))


def kernel(x, edge_index, batch, W1a, W2a, b1, W1b, W2b, b2,
           Wl1, bl1, Wl2, bl2):
    src = edge_index[0].reshape(_NW, _NG, _SG, _K)
    dst = edge_index[1].reshape(_NW, _NG, _SG, _K)

    sc_agg = _make_sc_agg()
    cntp = _make_sc_cnt()(dst).reshape(_NC, _N, _D)
    (agg1,) = sc_agg(x, src, dst)
    agg1 = agg1.reshape(_NC, _N, _D)

    h1 = pl.pallas_call(
        _tc_layer1_body,
        grid=(_N // _BN,),
        in_specs=[_row_spec(_D), _row_spec(_D), _row_spec(_D),
                  _row_spec(_D), _row_spec(_D),
                  _full_spec((_D, _D)), _full_spec((_D, _D)),
                  _full_spec((1, _D))],
        out_specs=_row_spec(_D),
        out_shape=jax.ShapeDtypeStruct((_N, _D), _f32),
    )(x, agg1[0], agg1[1], cntp[0], cntp[1], W1a, W2a, b1.reshape(1, _D))

    (agg2,) = sc_agg(h1, src, dst)
    agg2 = agg2.reshape(_NC, _N, _D)

    out = pl.pallas_call(
        _tc_layer2_body,
        grid=(_N // _BN,),
        in_specs=[_row_spec(_D), _row_spec(_D), _row_spec(_D),
                  _row_spec(_D), _row_spec(_D), _row_spec(1),
                  _full_spec((_D, _D)), _full_spec((_D, _D)),
                  _full_spec((1, _D)),
                  _full_spec((_D, _D // 2)), _full_spec((1, _D // 2)),
                  _full_spec((_D // 2, _OUT)), _full_spec((1, _OUT))],
        out_specs=_full_spec((_B, _OUT)),
        out_shape=jax.ShapeDtypeStruct((_B, _OUT), _f32),
        scratch_shapes=[pltpu.VMEM((_B, _D), _f32),
                        pltpu.VMEM((_B, 1), _f32)],
    )(h1, agg2[0], agg2[1], cntp[0], cntp[1], batch.reshape(_N, 1),
      W1b, W2b, b2.reshape(1, _D), Wl1, bl1.reshape(1, _D // 2),
      Wl2, bl2.reshape(1, _OUT))
    return out


# speedup vs baseline: 1.0653x; 1.0653x over previous
import jax
import jax.numpy as jnp
from jax import lax
from jax.experimental import pallas as pl
from jax.experimental.pallas import tpu as pltpu
from jax.experimental.pallas import tpu_sc as plsc

_N = 10000
_E = 320000
_D = 128
_B = 64
_OUT = 24

_NC = 2
_NS = 16
_NW = _NC * _NS
_EW = _E // _NW
_K = 100
_NCH = _EW // _K
_SG = 25
_NB = 3
_NG = _NCH // _SG
_RPT = _N // _NS

_f32 = jnp.float32

_MESH = plsc.VectorSubcoreMesh(core_axis_name="c", subcore_axis_name="s",
                               num_cores=_NC, num_subcores=_NS)


def _zero_fill(buf, nrows):
    def zrow(i, _):
        def zcol(j, _):
            buf[i, pl.ds(j * 16, 16)] = jnp.zeros((16,), _f32)
            return 0
        return lax.fori_loop(0, _D // 16, zcol, 0)
    lax.fori_loop(0, nrows, zrow, 0)


def _zero_acc_slice(rows, acc, r0):
    nfull, tail = _RPT // _K, _RPT % _K
    for t in range(nfull):
        pltpu.sync_copy(rows, acc.at[pl.ds(r0 + t * _K, _K)])
    if tail:
        pltpu.sync_copy(rows.at[pl.ds(0, tail)],
                        acc.at[pl.ds(r0 + nfull * _K, tail)])


def _make_sc_agg():
    def body(*refs):
        (h_hbm, src_hbm, dst_hbm, out_acc) = refs[:4]
        rows = refs[4:4 + _NB]
        sidx, didx, acc = refs[4 + _NB:7 + _NB]
        gsem = refs[7 + _NB:7 + 2 * _NB]
        ssem = refs[7 + 2 * _NB:7 + 3 * _NB]
        rows0 = rows[0]
        cid = lax.axis_index("c")
        sid = lax.axis_index("s")
        wid = cid * _NS + sid
        r0 = sid * _RPT

        _zero_fill(rows0, _K)
        _zero_acc_slice(rows0, acc, r0)

        plsc.subcore_barrier()

        def group(g, _):
            pltpu.sync_copy(src_hbm.at[wid, g], sidx)
            pltpu.sync_copy(dst_hbm.at[wid, g], didx)
            for pb in range(_NB - 1):
                pltpu.async_copy(h_hbm.at[sidx.at[pb]], rows[pb], gsem[pb])

            def step(c, _):
                def consume(b):
                    p = (b + _NB - 1) % _NB
                    pltpu.make_async_copy(
                        h_hbm.at[sidx.at[c]], rows[b], gsem[b]).wait()

                    @pl.when(c >= 1)
                    def _drain():
                        pltpu.make_async_copy(
                            rows[p], acc.at[didx.at[c]], ssem[p]).wait()

                    @pl.when(c + _NB - 1 < _SG)
                    def _prefetch():
                        pltpu.async_copy(
                            h_hbm.at[sidx.at[c + _NB - 1]], rows[p], gsem[p])
                    pltpu.async_copy(rows[b], acc.at[didx.at[c]], ssem[b],
                                     add=True)

                for b in range(_NB):
                    @pl.when(c % _NB == b)
                    def _go(b=b):
                        consume(b)
                return 0
            lax.fori_loop(0, _SG, step, 0)
            last = (_SG - 1) % _NB
            pltpu.make_async_copy(
                rows[last], acc.at[didx.at[0]], ssem[last]).wait()
            return 0
        lax.fori_loop(0, _NG, group, 0)

        plsc.subcore_barrier()
        pltpu.sync_copy(acc.at[pl.ds(r0, _RPT)], out_acc.at[cid, sid])

    return pl.kernel(
        body,
        out_type=[jax.ShapeDtypeStruct((_NC, _NS, _RPT, _D), _f32)],
        mesh=_MESH,
        scratch_types=(
            [pltpu.VMEM((_K, _D), _f32)] * _NB
            + [pltpu.VMEM((_SG, _K), jnp.int32),
               pltpu.VMEM((_SG, _K), jnp.int32),
               pltpu.VMEM_SHARED((_N, _D), _f32)]
            + [pltpu.SemaphoreType.DMA] * (2 * _NB)))


def _make_sc_cnt():
    def body(dst_hbm, out_cnt, ones_b, didx, acc, sem):
        cid = lax.axis_index("c")
        sid = lax.axis_index("s")
        wid = cid * _NS + sid
        r0 = sid * _RPT

        _zero_fill(ones_b, _K)
        _zero_acc_slice(ones_b, acc, r0)

        def orow(i, _):
            def ocol(j, _):
                ones_b[i, pl.ds(j * 16, 16)] = jnp.ones((16,), _f32)
                return 0
            return lax.fori_loop(0, _D // 16, ocol, 0)
        lax.fori_loop(0, _K, orow, 0)
        plsc.subcore_barrier()

        def group(g, _):
            pltpu.sync_copy(dst_hbm.at[wid, g], didx)

            def step(c, _):
                pltpu.async_copy(ones_b, acc.at[didx.at[c]], sem, add=True)

                @pl.when(c >= 3)
                def _drain():
                    pltpu.make_async_copy(
                        ones_b, acc.at[didx.at[c]], sem).wait()
                return 0
            lax.fori_loop(0, _SG, step, 0)
            for _t in range(3):
                pltpu.make_async_copy(
                    ones_b, acc.at[didx.at[0]], sem).wait()
            return 0
        lax.fori_loop(0, _NG, group, 0)

        plsc.subcore_barrier()
        pltpu.sync_copy(acc.at[pl.ds(r0, _RPT)], out_cnt.at[cid, sid])

    return pl.kernel(
        body,
        out_type=jax.ShapeDtypeStruct((_NC, _NS, _RPT, _D), _f32),
        mesh=_MESH,
        scratch_types=[
            pltpu.VMEM((_K, _D), _f32),
            pltpu.VMEM((_SG, _K), jnp.int32),
            pltpu.VMEM_SHARED((_N, _D), _f32),
            pltpu.SemaphoreType.DMA,
        ])


_BN = 1000


def _inv_count(c0, c1):
    return 1.0 / jnp.maximum(c0[:, 0:1] + c1[:, 0:1], 1.0)


def _tc_layer1_body(x, p0, p1, c0, c1, w1, w2, b, o):
    mean = (p0[...] + p1[...]) * _inv_count(c0[...], c1[...])
    h = (jnp.dot(x[...], w1[...], preferred_element_type=_f32)
         + jnp.dot(mean, w2[...], preferred_element_type=_f32) + b[...])
    o[...] = jnp.maximum(h, 0.0)


def _tc_layer2_body(h1, p0, p1, c0, c1, bat, w1, w2, b, wl1, bl1, wl2, bl2,
                    o, psum, pcnt):
    i = pl.program_id(0)

    @pl.when(i == 0)
    def _init():
        psum[...] = jnp.zeros((_B, _D), _f32)
        pcnt[...] = jnp.zeros((_B, 1), _f32)

    mean = (p0[...] + p1[...]) * _inv_count(c0[...], c1[...])
    h2 = (jnp.dot(h1[...], w1[...], preferred_element_type=_f32)
          + jnp.dot(mean, w2[...], preferred_element_type=_f32) + b[...])
    h2 = jnp.maximum(h2, 0.0)

    onehot = (bat[...] == lax.broadcasted_iota(jnp.int32, (_BN, _B), 1)
              ).astype(_f32)
    psum[...] += lax.dot_general(
        onehot, h2, (((0,), (0,)), ((), ())), preferred_element_type=_f32)
    pcnt[...] += jnp.sum(onehot, axis=0).reshape(_B, 1)

    @pl.when(i == pl.num_programs(0) - 1)
    def _head():
        pooled = psum[...] / jnp.maximum(pcnt[...], 1.0)
        z = jnp.maximum(
            jnp.dot(pooled, wl1[...], preferred_element_type=_f32) + bl1[...],
            0.0)
        o[...] = jnp.dot(z, wl2[...], preferred_element_type=_f32) + bl2[...]


def _row_spec(w):
    return pl.BlockSpec((_BN, w), lambda i: (i, 0))


def _full_spec(shape):
    return pl.BlockSpec(shape, lambda i: (0,) * len(shape))


def kernel(x, edge_index, batch, W1a, W2a, b1, W1b, W2b, b2,
           Wl1, bl1, Wl2, bl2):
    src = edge_index[0].reshape(_NW, _NG, _SG, _K)
    dst = edge_index[1].reshape(_NW, _NG, _SG, _K)

    sc_agg = _make_sc_agg()
    cntp = _make_sc_cnt()(dst).reshape(_NC, _N, _D)
    (agg1,) = sc_agg(x, src, dst)
    agg1 = agg1.reshape(_NC, _N, _D)

    h1 = pl.pallas_call(
        _tc_layer1_body,
        grid=(_N // _BN,),
        in_specs=[_row_spec(_D), _row_spec(_D), _row_spec(_D),
                  _row_spec(_D), _row_spec(_D),
                  _full_spec((_D, _D)), _full_spec((_D, _D)),
                  _full_spec((1, _D))],
        out_specs=_row_spec(_D),
        out_shape=jax.ShapeDtypeStruct((_N, _D), _f32),
    )(x, agg1[0], agg1[1], cntp[0], cntp[1], W1a, W2a, b1.reshape(1, _D))

    (agg2,) = sc_agg(h1, src, dst)
    agg2 = agg2.reshape(_NC, _N, _D)

    out = pl.pallas_call(
        _tc_layer2_body,
        grid=(_N // _BN,),
        in_specs=[_row_spec(_D), _row_spec(_D), _row_spec(_D),
                  _row_spec(_D), _row_spec(_D), _row_spec(1),
                  _full_spec((_D, _D)), _full_spec((_D, _D)),
                  _full_spec((1, _D)),
                  _full_spec((_D, _D // 2)), _full_spec((1, _D // 2)),
                  _full_spec((_D // 2, _OUT)), _full_spec((1, _OUT))],
        out_specs=_full_spec((_B, _OUT)),
        out_shape=jax.ShapeDtypeStruct((_B, _OUT), _f32),
        scratch_shapes=[pltpu.VMEM((_B, _D), _f32),
                        pltpu.VMEM((_B, 1), _f32)],
    )(h1, agg2[0], agg2[1], cntp[0], cntp[1], batch.reshape(_N, 1),
      W1b, W2b, b2.reshape(1, _D), Wl1, bl1.reshape(1, _D // 2),
      Wl2, bl2.reshape(1, _OUT))
    return out
